# Initial kernel scaffold; baseline (speedup 1.0000x reference)
import jax
import jax.numpy as jnp
from jax.experimental import pallas as pl


def kernel(u_emb, p_emb, p_views, t, user_ids, prop_ids):
    raise NotImplementedError("write your pallas kernel here")


# speedup vs baseline: 8.7896x; 8.7896x over previous
import functools

import jax
import jax.numpy as jnp
from jax import lax
from jax.experimental import pallas as pl
from jax.experimental.pallas import tpu as pltpu

_B = 1024
_NU = 128
_NP = 512
_TEMP = 0.3
_LOW = 0.4
_HIGH = 0.7
_MARGIN = 0.1
_LAMBDA_ORTHO = 0.1
_CHUNK = 128
_NCHUNK = _B // _CHUNK


def _loss_body(u_ref, p_ref, ut_ref, pt_ref, t_row_ref, t_col_ref,
               uid_row_ref, uid_col_ref, pid_row_ref, pid_col_ref,
               out_ref, wcol_ref, dcol_ref):
    f32 = jnp.float32
    t_row = t_row_ref[...]
    t_col = t_col_ref[...]
    uid_row = uid_row_ref[...]
    uid_col = uid_col_ref[...]
    pid_row = pid_row_ref[...]
    pid_col = pid_col_ref[...]

    diff = u_ref[...] - p_ref[...] + 1e-6
    dcol_ref[...] = jnp.sqrt(jnp.sum(diff * diff, axis=1, keepdims=True))
    difft = ut_ref[...] - pt_ref[...] + 1e-6
    dist_row = jnp.sqrt(jnp.sum(difft * difft, axis=0, keepdims=True))

    def pass_a(ib, acc):
        key_loser_row, p_loser_row, u_loser_row = acc
        base = ib * _CHUNK
        uc = uid_col_ref[pl.ds(base, _CHUNK), :]
        pc = pid_col_ref[pl.ds(base, _CHUNK), :]
        same_u = (uc == uid_row)
        same_p = (pc == pid_row)
        same_k = same_u & same_p
        i_g = lax.broadcasted_iota(jnp.int32, (_CHUNK, _B), 0) + base
        j_g = lax.broadcasted_iota(jnp.int32, (_CHUNK, _B), 1)
        jgt = j_g > i_g
        igt = i_g > j_g
        loser_c = jnp.max((same_k & jgt).astype(f32), axis=1, keepdims=True)
        wcol_ref[pl.ds(base, _CHUNK), :] = 1.0 - loser_c
        key_loser_row = jnp.maximum(
            key_loser_row, jnp.max((same_k & igt).astype(f32), axis=0, keepdims=True))
        p_loser_row = jnp.maximum(
            p_loser_row, jnp.max((same_p & igt).astype(f32), axis=0, keepdims=True))
        u_loser_row = jnp.maximum(
            u_loser_row, jnp.max((same_u & igt).astype(f32), axis=0, keepdims=True))
        return key_loser_row, p_loser_row, u_loser_row

    zrow = jnp.zeros((1, _B), dtype=f32)
    key_loser_row, p_loser_row, u_loser_row = lax.fori_loop(
        0, _NCHUNK, pass_a, (zrow, zrow, zrow))
    winner_row = 1.0 - key_loser_row
    nP = jnp.sum(1.0 - p_loser_row)
    Ucnt = jnp.sum(1.0 - u_loser_row)

    def pass_b(ib, hacc):
        base = ib * _CHUNK
        uc = uid_col_ref[pl.ds(base, _CHUNK), :]
        tc = t_col_ref[pl.ds(base, _CHUNK), :]
        dc = dcol_ref[pl.ds(base, _CHUNK), :]
        wc = wcol_ref[pl.ds(base, _CHUNK), :]
        i_g = lax.broadcasted_iota(jnp.int32, (_CHUNK, _B), 0) + base
        j_g = lax.broadcasted_iota(jnp.int32, (_CHUNK, _B), 1)
        term = jax.nn.relu(jnp.sign(t_row - tc) * (dc - dist_row) + _MARGIN)
        mask = ((uc == uid_row) & (j_g > i_g)
                & (tc != t_row) & (tc > 0.0) & (t_row > 0.0))
        contrib = term * mask.astype(f32) * wc * winner_row
        return hacc + jnp.sum(contrib, axis=0, keepdims=True)

    hinge_vec = lax.fori_loop(0, _NCHUNK, pass_b, zrow)
    hinge = jnp.sum(hinge_vec) / Ucnt

    e_row = jnp.exp(-dist_row / _TEMP)
    pos_row = (t_row > _HIGH).astype(f32)
    w_row = jnp.where(t_row > _HIGH, 1.0,
                      jnp.where(t_row < _LOW, 1.5, 0.3))
    onehot = (lax.broadcasted_iota(jnp.int32, (_NU, _B), 0)
              == uid_row).astype(f32)
    k_u = jnp.sum(onehot * winner_row, axis=1, keepdims=True)
    sum_we = jnp.sum(onehot * (winner_row * w_row * e_row), axis=1, keepdims=True)
    num = jnp.sum(onehot * (winner_row * pos_row * e_row), axis=1, keepdims=True)
    npos = jnp.sum(onehot * (winner_row * pos_row), axis=1, keepdims=True)
    denom = 0.3 * (nP - k_u) + sum_we + 1e-8
    valid = (npos > 0.0).astype(f32)
    num_safe = jnp.where(npos > 0.0, num, denom)
    lpu = -jnp.log(num_safe / denom)
    n_valid = jnp.sum(valid)
    nce = jnp.where(n_valid > 0.0,
                    jnp.sum(lpu * valid) / jnp.maximum(n_valid, 1.0), 0.0)

    gram = jnp.dot(ut_ref[...], p_ref[...], preferred_element_type=f32)
    ortho = jnp.mean(jnp.abs(gram))

    total = nce + hinge + ortho * _LAMBDA_ORTHO
    out_ref[...] = jnp.reshape(total, (1, 1))


@functools.partial(jax.jit, static_argnames=("interpret",))
def _run(u_emb, p_emb, t, uid, pid, interpret=False):
    call = pl.pallas_call(
        _loss_body,
        out_shape=jax.ShapeDtypeStruct((1, 1), jnp.float32),
        scratch_shapes=[pltpu.VMEM((_B, 1), jnp.float32),
                        pltpu.VMEM((_B, 1), jnp.float32)],
        interpret=interpret,
    )
    out = call(u_emb, p_emb,
               u_emb.T, p_emb.T,
               t.reshape(1, _B), t.reshape(_B, 1),
               uid.reshape(1, _B), uid.reshape(_B, 1),
               pid.reshape(1, _B), pid.reshape(_B, 1))
    return out[0, 0]


def kernel(u_emb, p_emb, p_views, t, user_ids, prop_ids):
    del p_views
    return _run(u_emb, p_emb, t.astype(jnp.float32),
                user_ids.astype(jnp.int32), prop_ids.astype(jnp.int32))
